# Initial kernel scaffold; baseline (speedup 1.0000x reference)
import jax
import jax.numpy as jnp
import numpy as np
from jax import lax
from jax.experimental import pallas as pl
from jax.experimental.pallas import tpu as pltpu
from jax.experimental.pallas import tpu_sc as plsc

N_ATOMS = 50000
N_EDGES = 800000
NWAVE = 8
NANG = 9
NCON = 64
CUTOFF = 4.0
INDEX_L = (0, 1, 1, 1, 2, 2, 2, 2, 2)

NC = 2
NS = 16
LANES = 16
NW = NC * NS

CHUNK = 128
E_PAD = 802816
N_CHUNKS = E_PAD // CHUNK
A_HALF = N_ATOMS // NC
A_ROWS = 25088
TRASH = A_HALF

BA = 5000
BE = 4096
N_ABLK = N_ATOMS // BA
N_EBLK = E_PAD // BE


def _act(x):
    return x * jax.nn.sigmoid(x)


def _layer_norm(x):
    m = jnp.mean(x, axis=-1, keepdims=True)
    v = jnp.var(x, axis=-1, keepdims=True)
    return (x - m) * lax.rsqrt(v + 1e-5)


def _coeff_body(sp_ref, cw0, cb0, cw1, cb1, cwf, cbf, nw0, nb0, nw1, nb1, nwf,
                nbf, lc_ref, emb_ref):
    x = sp_ref[...]

    def mlp(w0, b0, w1, b1, wf, bf):
        h = _act(_layer_norm(x * w0[...] + b0[...]))
        h = h + _act(_layer_norm(
            jnp.dot(h, w1[...], preferred_element_type=jnp.float32) + b1[...]))
        return jnp.dot(h, wf[...], preferred_element_type=jnp.float32) + bf[...]

    lc_ref[...] = mlp(cw0, cb0, cw1, cb1, cwf, cbf)
    nb = mlp(nw0, nb0, nw1, nb1, nwf, nbf)
    emb_ref[...] = jnp.concatenate(
        [nb, jnp.zeros((nb.shape[0], 8), jnp.float32)], axis=1)


def _edge_orbital(dv, emb):
    x = dv[:, 0:1]
    y = dv[:, 1:2]
    z = dv[:, 2:3]
    r2 = x * x + y * y + z * z
    r = jnp.sqrt(r2)
    a = emb[:, 0:NWAVE]
    b = emb[:, NWAVE:2 * NWAVE]
    c0 = emb[:, 2 * NWAVE:3 * NWAVE]
    cut = jnp.square(0.5 * jnp.cos(r * (np.pi / CUTOFF)) + 0.5)
    radial = jnp.exp(-jnp.square(b * (r - c0)))
    rad8 = cut * radial * a
    sph = [
        0.28209479177387814 * jnp.ones_like(x),
        0.4886025119029199 * y,
        0.4886025119029199 * z,
        0.4886025119029199 * x,
        1.0925484305920792 * x * y,
        1.0925484305920792 * y * z,
        0.31539156525252005 * (3.0 * z * z - r2),
        1.0925484305920792 * x * z,
        0.5462742152960396 * (x * x - y * y),
    ]
    return jnp.concatenate([s * rad8 for s in sph], axis=1)


def _edge_fwd_body(cc_ref, cn_ref, emb_ref, orb_ref):
    dv = cn_ref[:, 0:3] - cc_ref[:, 0:3]
    orb = _edge_orbital(dv, emb_ref[...])
    orb_ref[...] = jnp.concatenate(
        [orb, jnp.zeros((orb.shape[0], 8), jnp.float32)], axis=1)


def _edge_bwd_body(cc_ref, cn_ref, emb_ref, gorb_ref, sn_ref, sc_ref):
    dv = cn_ref[:, 0:3] - cc_ref[:, 0:3]
    emb = emb_ref[...]
    _, vjp_fn = jax.vjp(lambda d: _edge_orbital(d, emb), dv)
    (g_dv,) = vjp_fn(gorb_ref[:, 0:72])
    pad = jnp.zeros((g_dv.shape[0], 13), jnp.float32)
    sn_ref[...] = jnp.concatenate([-g_dv, pad], axis=1)
    sc_ref[...] = jnp.concatenate([g_dv, pad], axis=1)


def _atom_body(co_ref, lc_ref, cc_ref, w0, b0, w1, b1, wf, bf,
               out_ref, gco_ref, en_ref):
    lc = lc_ref[...]
    cc = cc_ref[...]

    def f(co72):
        s = jnp.zeros((co72.shape[0], NCON), jnp.float32)
        for k in range(NANG):
            ck = jnp.dot(co72[:, 8 * k:8 * k + 8], cc[8 * k:8 * k + 8, :],
                         preferred_element_type=jnp.float32)
            s = s + ck * ck
        dens = s * lc
        h = _act(_layer_norm(
            jnp.dot(dens, w0[...], preferred_element_type=jnp.float32) + b0[...]))
        h = h + _act(_layer_norm(
            jnp.dot(h, w1[...], preferred_element_type=jnp.float32) + b1[...]))
        outp = jnp.dot(h, wf[...], preferred_element_type=jnp.float32) + bf[...]
        return jnp.sum(outp), outp

    (e_part, outp), vjp_fn, _ = jax.vjp(f, co_ref[:, 0:72], has_aux=False)
    (g_co,) = vjp_fn((jnp.float32(1.0), jnp.zeros_like(outp)))
    out_ref[...] = outp
    gco_ref[...] = jnp.concatenate(
        [g_co, jnp.zeros((g_co.shape[0], 8), jnp.float32)], axis=1)

    @pl.when(pl.program_id(0) == 0)
    def _():
        en_ref[0, 0] = 0.0

    en_ref[0, 0] += e_part


_MESH = plsc.VectorSubcoreMesh(core_axis_name="c", subcore_axis_name="s")


def _sc_gather3(cart_hbm, coeff_hbm, ci_hbm, ni_hbm, si_hbm,
                outc_hbm, outn_hbm, oute_hbm,
                ic_v, in_v, is_v, bc_v, bn_v, be_v, sem0, sem1, sem2):
    wid = lax.axis_index("s") * NC + lax.axis_index("c")

    @pl.loop(wid, N_CHUNKS, step=NW)
    def _(t):
        base = t * CHUNK
        pltpu.sync_copy(ci_hbm.at[pl.ds(base, CHUNK)], ic_v)
        pltpu.sync_copy(ni_hbm.at[pl.ds(base, CHUNK)], in_v)
        pltpu.sync_copy(si_hbm.at[pl.ds(base, CHUNK)], is_v)
        d0 = pltpu.async_copy(cart_hbm.at[ic_v], bc_v, sem0)
        d1 = pltpu.async_copy(cart_hbm.at[in_v], bn_v, sem1)
        d2 = pltpu.async_copy(coeff_hbm.at[is_v], be_v, sem2)
        d0.wait()
        d1.wait()
        d2.wait()
        pltpu.sync_copy(bc_v, outc_hbm.at[pl.ds(base, CHUNK)])
        pltpu.sync_copy(bn_v, outn_hbm.at[pl.ds(base, CHUNK)])
        pltpu.sync_copy(be_v, oute_hbm.at[pl.ds(base, CHUNK)])


def _sc_gather1(tab_hbm, ci_hbm, out_hbm, i_v, b_v, sem0):
    wid = lax.axis_index("s") * NC + lax.axis_index("c")

    @pl.loop(wid, N_CHUNKS, step=NW)
    def _(t):
        base = t * CHUNK
        pltpu.sync_copy(ci_hbm.at[pl.ds(base, CHUNK)], i_v)
        pltpu.async_copy(tab_hbm.at[i_v], b_v, sem0).wait()
        pltpu.sync_copy(b_v, out_hbm.at[pl.ds(base, CHUNK)])


def _zero_vmem(zv, width):
    zrow = jnp.zeros((1, LANES), jnp.float32)

    @pl.loop(0, zv.shape[0])
    def _(r):
        for c in range(width // LANES):
            zv[pl.ds(r, 1), pl.ds(c * LANES, LANES)] = zrow


def _compute_local_idx(idx_v, li_v, core):
    base = core * A_HALF
    for i in range(CHUNK // LANES):
        v = idx_v[pl.ds(i * LANES, LANES)]
        li = v - base
        ok = (li >= 0) & (li < A_HALF)
        li_v[pl.ds(i * LANES, LANES)] = jnp.where(ok, li, TRASH)


def _sc_scatter_orb(orb_hbm, si_hbm, out_hbm, acc_sh, idx_v, li_v, orb_v, zero_v):
    core = lax.axis_index("c")
    sid = lax.axis_index("s")
    _zero_vmem(zero_v, 80)

    @pl.loop(sid, A_ROWS // CHUNK, step=NS)
    def _(c):
        pltpu.sync_copy(zero_v, acc_sh.at[pl.ds(c * CHUNK, CHUNK)])

    plsc.subcore_barrier()

    @pl.loop(sid, N_CHUNKS, step=NS)
    def _(t):
        base = t * CHUNK
        pltpu.sync_copy(si_hbm.at[pl.ds(base, CHUNK)], idx_v)
        pltpu.sync_copy(orb_hbm.at[pl.ds(base, CHUNK)], orb_v)
        _compute_local_idx(idx_v, li_v, core)
        pltpu.sync_copy(orb_v, acc_sh.at[li_v], add=True)

    plsc.subcore_barrier()

    @pl.loop(sid, 125, step=NS)
    def _(c):
        r0 = c * 200
        pltpu.sync_copy(acc_sh.at[pl.ds(r0, 200)],
                        out_hbm.at[pl.ds(core * A_HALF + r0, 200)])


def _sc_scatter_force(sn_hbm, sc_hbm, ni_hbm, ci_hbm, out_hbm, acc_sh,
                      idxn_v, idxc_v, lin_v, lic_v, bn_v, bc_v, zero_v):
    core = lax.axis_index("c")
    sid = lax.axis_index("s")
    _zero_vmem(zero_v, 16)

    @pl.loop(sid, A_ROWS // CHUNK, step=NS)
    def _(c):
        pltpu.sync_copy(zero_v, acc_sh.at[pl.ds(c * CHUNK, CHUNK)])

    plsc.subcore_barrier()

    @pl.loop(sid, N_CHUNKS, step=NS)
    def _(t):
        base = t * CHUNK
        pltpu.sync_copy(ni_hbm.at[pl.ds(base, CHUNK)], idxn_v)
        pltpu.sync_copy(ci_hbm.at[pl.ds(base, CHUNK)], idxc_v)
        pltpu.sync_copy(sn_hbm.at[pl.ds(base, CHUNK)], bn_v)
        pltpu.sync_copy(sc_hbm.at[pl.ds(base, CHUNK)], bc_v)
        _compute_local_idx(idxn_v, lin_v, core)
        _compute_local_idx(idxc_v, lic_v, core)
        pltpu.sync_copy(bn_v, acc_sh.at[lin_v], add=True)
        pltpu.sync_copy(bc_v, acc_sh.at[lic_v], add=True)

    plsc.subcore_barrier()

    @pl.loop(sid, 125, step=NS)
    def _(c):
        r0 = c * 200
        pltpu.sync_copy(acc_sh.at[pl.ds(r0, 200)],
                        out_hbm.at[pl.ds(core * A_HALF + r0, 200)])


def _full(shape):
    return pl.BlockSpec(shape, lambda i: (0,) * len(shape))


def _coeff_call(species, p):
    specs = [pl.BlockSpec((BA, 1), lambda i: (i, 0))]
    args = [species]
    for pre, wfw in (("center", NCON), ("neigh", 3 * NWAVE)):
        for nm, shp in (("W0", (1, 8)), ("b0", (1, 8)), ("W1", (8, 8)),
                        ("b1", (1, 8)), ("Wf", (8, wfw)), ("bf", (1, wfw))):
            a = p[pre + "_" + nm].reshape(shp)
            specs.append(_full(shp))
            args.append(a)
    return pl.pallas_call(
        _coeff_body,
        grid=(N_ABLK,),
        in_specs=specs,
        out_specs=[pl.BlockSpec((BA, NCON), lambda i: (i, 0)),
                   pl.BlockSpec((BA, 32), lambda i: (i, 0))],
        out_shape=[jax.ShapeDtypeStruct((N_ATOMS, NCON), jnp.float32),
                   jax.ShapeDtypeStruct((N_ATOMS, 32), jnp.float32)],
    )(*args)


def _edge_fwd_call(cart_c, cart_n, emb):
    return pl.pallas_call(
        _edge_fwd_body,
        grid=(N_EBLK,),
        in_specs=[pl.BlockSpec((BE, 16), lambda i: (i, 0)),
                  pl.BlockSpec((BE, 16), lambda i: (i, 0)),
                  pl.BlockSpec((BE, 32), lambda i: (i, 0))],
        out_specs=pl.BlockSpec((BE, 80), lambda i: (i, 0)),
        out_shape=jax.ShapeDtypeStruct((E_PAD, 80), jnp.float32),
    )(cart_c, cart_n, emb)


def _edge_bwd_call(cart_c, cart_n, emb, gorb):
    return pl.pallas_call(
        _edge_bwd_body,
        grid=(N_EBLK,),
        in_specs=[pl.BlockSpec((BE, 16), lambda i: (i, 0)),
                  pl.BlockSpec((BE, 16), lambda i: (i, 0)),
                  pl.BlockSpec((BE, 32), lambda i: (i, 0)),
                  pl.BlockSpec((BE, 80), lambda i: (i, 0))],
        out_specs=[pl.BlockSpec((BE, 16), lambda i: (i, 0)),
                   pl.BlockSpec((BE, 16), lambda i: (i, 0))],
        out_shape=[jax.ShapeDtypeStruct((E_PAD, 16), jnp.float32),
                   jax.ShapeDtypeStruct((E_PAD, 16), jnp.float32)],
    )(cart_c, cart_n, emb, gorb)


def _atom_call(center_orbital, local_coeff, cc, p):
    args = [center_orbital, local_coeff, cc]
    specs = [pl.BlockSpec((BA, 80), lambda i: (i, 0)),
             pl.BlockSpec((BA, NCON), lambda i: (i, 0)),
             _full((72, NCON))]
    for nm, shp in (("W0", (NCON, 64)), ("b0", (1, 64)), ("W1", (64, 64)),
                    ("b1", (1, 64)), ("Wf", (64, 1)), ("bf", (1, 1))):
        args.append(p["out_" + nm].reshape(shp))
        specs.append(_full(shp))
    return pl.pallas_call(
        _atom_body,
        grid=(N_ABLK,),
        in_specs=specs,
        out_specs=[pl.BlockSpec((BA, 1), lambda i: (i, 0)),
                   pl.BlockSpec((BA, 80), lambda i: (i, 0)),
                   pl.BlockSpec((1, 1), lambda i: (0, 0))],
        out_shape=[jax.ShapeDtypeStruct((N_ATOMS, 1), jnp.float32),
                   jax.ShapeDtypeStruct((N_ATOMS, 80), jnp.float32),
                   jax.ShapeDtypeStruct((1, 1), jnp.float32)],
    )(*args)


def _gather3_call(cart_pad, coeff_pad, ci, ni, si):
    f = pl.kernel(
        _sc_gather3,
        out_type=[jax.ShapeDtypeStruct((E_PAD, 16), jnp.float32),
                  jax.ShapeDtypeStruct((E_PAD, 16), jnp.float32),
                  jax.ShapeDtypeStruct((E_PAD, 32), jnp.float32)],
        mesh=_MESH,
        scratch_types=[pltpu.VMEM((CHUNK,), jnp.int32),
                       pltpu.VMEM((CHUNK,), jnp.int32),
                       pltpu.VMEM((CHUNK,), jnp.int32),
                       pltpu.VMEM((CHUNK, 16), jnp.float32),
                       pltpu.VMEM((CHUNK, 16), jnp.float32),
                       pltpu.VMEM((CHUNK, 32), jnp.float32),
                       pltpu.SemaphoreType.DMA,
                       pltpu.SemaphoreType.DMA,
                       pltpu.SemaphoreType.DMA],
    )
    return f(cart_pad, coeff_pad, ci, ni, si)


def _gather1_call(table, ci):
    f = pl.kernel(
        _sc_gather1,
        out_type=jax.ShapeDtypeStruct((E_PAD, 80), jnp.float32),
        mesh=_MESH,
        scratch_types=[pltpu.VMEM((CHUNK,), jnp.int32),
                       pltpu.VMEM((CHUNK, 80), jnp.float32),
                       pltpu.SemaphoreType.DMA],
    )
    return f(table, ci)


def _scatter_orb_call(orb, si):
    f = pl.kernel(
        _sc_scatter_orb,
        out_type=jax.ShapeDtypeStruct((N_ATOMS, 80), jnp.float32),
        mesh=_MESH,
        scratch_types=[pltpu.VMEM_SHARED((A_ROWS, 80), jnp.float32),
                       pltpu.VMEM((CHUNK,), jnp.int32),
                       pltpu.VMEM((CHUNK,), jnp.int32),
                       pltpu.VMEM((CHUNK, 80), jnp.float32),
                       pltpu.VMEM((CHUNK, 80), jnp.float32)],
    )
    return f(orb, si)


def _scatter_force_call(sn, sc, ni, ci):
    f = pl.kernel(
        _sc_scatter_force,
        out_type=jax.ShapeDtypeStruct((N_ATOMS, 16), jnp.float32),
        mesh=_MESH,
        scratch_types=[pltpu.VMEM_SHARED((A_ROWS, 16), jnp.float32),
                       pltpu.VMEM((CHUNK,), jnp.int32),
                       pltpu.VMEM((CHUNK,), jnp.int32),
                       pltpu.VMEM((CHUNK,), jnp.int32),
                       pltpu.VMEM((CHUNK,), jnp.int32),
                       pltpu.VMEM((CHUNK, 16), jnp.float32),
                       pltpu.VMEM((CHUNK, 16), jnp.float32),
                       pltpu.VMEM((CHUNK, 16), jnp.float32)],
    )
    return f(sn, sc, ni, ci)


def kernel(cart, centerlist, neighlist, local_species, neigh_species, nlocal,
           atom_species, params):
    pad_e = E_PAD - N_EDGES
    cart_pad = jnp.concatenate(
        [cart, jnp.zeros((N_ATOMS, 13), jnp.float32)], axis=1)

    def pad_idx(a, fill):
        return jnp.concatenate(
            [a.astype(jnp.int32), jnp.full((pad_e,), fill, jnp.int32)])

    ci_g = pad_idx(centerlist, 0)
    ni_g = pad_idx(neighlist, 0)
    si_g = pad_idx(local_species, 0)
    ci_s = pad_idx(centerlist, N_ATOMS)
    ni_s = pad_idx(neighlist, N_ATOMS)

    cc = params["contracted_coeff"][0][jnp.array(INDEX_L)].reshape(72, NCON)

    local_coeff, neigh_emb = _coeff_call(atom_species, params)
    cart_c, cart_n, emb = _gather3_call(cart_pad, neigh_emb, ci_g, ni_g, si_g)
    orb = _edge_fwd_call(cart_c, cart_n, emb)
    center_orbital = _scatter_orb_call(orb, ci_s)
    output, g_co, energy = _atom_call(center_orbital, local_coeff, cc, params)
    g_orb = _gather1_call(g_co, ci_g)
    sn, sc_ = _edge_bwd_call(cart_c, cart_n, emb, g_orb)
    neg_grad = _scatter_force_call(sn, sc_, ni_s, ci_s)

    force = neg_grad[:, 0:3].reshape(-1)
    return (energy.reshape(()), force, output)


# speedup vs baseline: 6.4753x; 6.4753x over previous
import jax
import jax.numpy as jnp
import numpy as np
from jax import lax
from jax.experimental import pallas as pl
from jax.experimental.pallas import tpu as pltpu
from jax.experimental.pallas import tpu_sc as plsc

N_ATOMS = 50000
N_EDGES = 800000
NWAVE = 8
NANG = 9
NCON = 64
CUTOFF = 4.0
INDEX_L = (0, 1, 1, 1, 2, 2, 2, 2, 2)

NC = 2
NS = 16
LANES = 16
NW = NC * NS

CHUNK = 128
SCH = 64
E_PAD = 802816
N_CHUNKS = E_PAD // CHUNK
N_SCH = E_PAD // SCH
A_HALF = N_ATOMS // NC
A_ROWS = 25024
TRASH = A_HALF

BA = 2000
BE = 2048
N_ABLK = N_ATOMS // BA
N_EBLK = E_PAD // BE


def _act(x):
    return x * jax.nn.sigmoid(x)


def _layer_norm(x):
    m = jnp.mean(x, axis=-1, keepdims=True)
    v = jnp.var(x, axis=-1, keepdims=True)
    return (x - m) * lax.rsqrt(v + 1e-5)


def _coeff_body(sp_ref, cw0, cb0, cw1, cb1, cwf, cbf, nw0, nb0, nw1, nb1, nwf,
                nbf, lc_ref, emb_ref):
    x = sp_ref[...]

    def mlp(w0, b0, w1, b1, wf, bf):
        h = _act(_layer_norm(x * w0[...] + b0[...]))
        h = h + _act(_layer_norm(
            jnp.dot(h, w1[...], preferred_element_type=jnp.float32) + b1[...]))
        return jnp.dot(h, wf[...], preferred_element_type=jnp.float32) + bf[...]

    lc_ref[...] = mlp(cw0, cb0, cw1, cb1, cwf, cbf)
    nb = mlp(nw0, nb0, nw1, nb1, nwf, nbf)
    emb_ref[...] = jnp.concatenate(
        [nb, jnp.zeros((nb.shape[0], 8), jnp.float32)], axis=1)


def _edge_orbital(dv, emb):
    x = dv[:, 0:1]
    y = dv[:, 1:2]
    z = dv[:, 2:3]
    r2 = x * x + y * y + z * z
    r = jnp.sqrt(r2)
    a = emb[:, 0:NWAVE]
    b = emb[:, NWAVE:2 * NWAVE]
    c0 = emb[:, 2 * NWAVE:3 * NWAVE]
    cut = jnp.square(0.5 * jnp.cos(r * (np.pi / CUTOFF)) + 0.5)
    radial = jnp.exp(-jnp.square(b * (r - c0)))
    rad8 = cut * radial * a
    sph = [
        0.28209479177387814 * jnp.ones_like(x),
        0.4886025119029199 * y,
        0.4886025119029199 * z,
        0.4886025119029199 * x,
        1.0925484305920792 * x * y,
        1.0925484305920792 * y * z,
        0.31539156525252005 * (3.0 * z * z - r2),
        1.0925484305920792 * x * z,
        0.5462742152960396 * (x * x - y * y),
    ]
    return jnp.concatenate([s * rad8 for s in sph], axis=1)


def _edge_fwd_body(cc_ref, cn_ref, emb_ref, orb_ref):
    dv = cn_ref[:, 0:3] - cc_ref[:, 0:3]
    orb = _edge_orbital(dv, emb_ref[...])
    orb_ref[...] = jnp.concatenate(
        [orb, jnp.zeros((orb.shape[0], 8), jnp.float32)], axis=1)


def _edge_bwd_body(cc_ref, cn_ref, emb_ref, gorb_ref, sn_ref, sc_ref):
    dv = cn_ref[:, 0:3] - cc_ref[:, 0:3]
    emb = emb_ref[...]
    _, vjp_fn = jax.vjp(lambda d: _edge_orbital(d, emb), dv)
    (g_dv,) = vjp_fn(gorb_ref[:, 0:72])
    pad = jnp.zeros((g_dv.shape[0], 13), jnp.float32)
    sn_ref[...] = jnp.concatenate([-g_dv, pad], axis=1)
    sc_ref[...] = jnp.concatenate([g_dv, pad], axis=1)


def _atom_body(co_ref, lc_ref, cc_ref, w0, b0, w1, b1, wf, bf,
               out_ref, gco_ref, en_ref):
    lc = lc_ref[...]
    cc = cc_ref[...]

    def f(co72):
        s = jnp.zeros((co72.shape[0], NCON), jnp.float32)
        for k in range(NANG):
            ck = jnp.dot(co72[:, 8 * k:8 * k + 8], cc[8 * k:8 * k + 8, :],
                         preferred_element_type=jnp.float32)
            s = s + ck * ck
        dens = s * lc
        h = _act(_layer_norm(
            jnp.dot(dens, w0[...], preferred_element_type=jnp.float32) + b0[...]))
        h = h + _act(_layer_norm(
            jnp.dot(h, w1[...], preferred_element_type=jnp.float32) + b1[...]))
        outp = jnp.dot(h, wf[...], preferred_element_type=jnp.float32) + bf[...]
        return jnp.sum(outp), outp

    (e_part, outp), vjp_fn = jax.vjp(f, co_ref[:, 0:72])
    (g_co,) = vjp_fn((jnp.float32(1.0), jnp.zeros_like(outp)))
    out_ref[...] = outp
    gco_ref[...] = jnp.concatenate(
        [g_co, jnp.zeros((g_co.shape[0], 8), jnp.float32)], axis=1)

    @pl.when(pl.program_id(0) == 0)
    def _():
        en_ref[...] = jnp.zeros((1, 1), jnp.float32)

    en_ref[...] = en_ref[...] + jnp.reshape(e_part, (1, 1))


import functools as _ft


@_ft.cache
def _mesh():
    return plsc.VectorSubcoreMesh(core_axis_name="c", subcore_axis_name="s")


_SC_PARAMS = pltpu.CompilerParams(use_tc_tiling_on_sc=False)


def _sc_gather3(cart_hbm, coeff_hbm, ci_hbm, ni_hbm, si_hbm,
                outc_hbm, outn_hbm, oute_hbm,
                ic_v, in_v, is_v, bc_v, bn_v, be_v, sem0, sem1, sem2):
    wid = lax.axis_index("s") * NC + lax.axis_index("c")

    @pl.loop(wid, N_CHUNKS, step=NW)
    def _(t):
        base = t * CHUNK
        pltpu.sync_copy(ci_hbm.at[pl.ds(base, CHUNK)], ic_v)
        pltpu.sync_copy(ni_hbm.at[pl.ds(base, CHUNK)], in_v)
        pltpu.sync_copy(si_hbm.at[pl.ds(base, CHUNK)], is_v)
        d0 = pltpu.async_copy(cart_hbm.at[ic_v], bc_v, sem0)
        d1 = pltpu.async_copy(cart_hbm.at[in_v], bn_v, sem1)
        d2 = pltpu.async_copy(coeff_hbm.at[is_v], be_v, sem2)
        d0.wait()
        d1.wait()
        d2.wait()
        pltpu.sync_copy(bc_v, outc_hbm.at[pl.ds(base, CHUNK)])
        pltpu.sync_copy(bn_v, outn_hbm.at[pl.ds(base, CHUNK)])
        pltpu.sync_copy(be_v, oute_hbm.at[pl.ds(base, CHUNK)])


def _sc_gather1(tab_hbm, ci_hbm, out_hbm, i_v, b_v, sem0):
    wid = lax.axis_index("s") * NC + lax.axis_index("c")

    @pl.loop(wid, N_CHUNKS, step=NW)
    def _(t):
        base = t * CHUNK
        pltpu.sync_copy(ci_hbm.at[pl.ds(base, CHUNK)], i_v)
        pltpu.async_copy(tab_hbm.at[i_v], b_v, sem0).wait()
        pltpu.sync_copy(b_v, out_hbm.at[pl.ds(base, CHUNK)])


def _zero_vmem(zv, width):
    zrow = jnp.zeros((1, LANES), jnp.float32)

    @pl.loop(0, zv.shape[0])
    def _(r):
        for c in range(width // LANES):
            zv[pl.ds(r, 1), pl.ds(c * LANES, LANES)] = zrow


def _compute_local_idx(idx_v, li_v, core):
    base = core * A_HALF
    for i in range(SCH // LANES):
        v = idx_v[pl.ds(i * LANES, LANES)]
        li = v - base
        ok = (li >= 0) & (li < A_HALF)
        li_v[pl.ds(i * LANES, LANES)] = jnp.where(ok, li, TRASH)


def _sc_scatter_orb(orb_hbm, si_hbm, out_hbm, acc_sh, idx_v, li_v, orb_v):
    core = lax.axis_index("c")
    sid = lax.axis_index("s")
    _zero_vmem(orb_v, 80)

    @pl.loop(sid, A_ROWS // SCH, step=NS)
    def _(c):
        pltpu.sync_copy(orb_v, acc_sh.at[pl.ds(c * SCH, SCH)])

    plsc.subcore_barrier()

    @pl.loop(sid, N_SCH, step=NS)
    def _(t):
        base = t * SCH
        pltpu.sync_copy(si_hbm.at[pl.ds(base, SCH)], idx_v)
        pltpu.sync_copy(orb_hbm.at[pl.ds(base, SCH)], orb_v)
        _compute_local_idx(idx_v, li_v, core)
        pltpu.sync_copy(orb_v, acc_sh.at[li_v], add=True)

    plsc.subcore_barrier()

    @pl.loop(sid, 125, step=NS)
    def _(c):
        r0 = c * 200
        pltpu.sync_copy(acc_sh.at[pl.ds(r0, 200)],
                        out_hbm.at[pl.ds(core * A_HALF + r0, 200)])


def _sc_scatter_force(sn_hbm, sc_hbm, ni_hbm, ci_hbm, out_hbm, acc_sh,
                      idxn_v, idxc_v, lin_v, lic_v, bn_v, bc_v):
    core = lax.axis_index("c")
    sid = lax.axis_index("s")
    _zero_vmem(bn_v, 16)

    @pl.loop(sid, A_ROWS // SCH, step=NS)
    def _(c):
        pltpu.sync_copy(bn_v, acc_sh.at[pl.ds(c * SCH, SCH)])

    plsc.subcore_barrier()

    @pl.loop(sid, N_SCH, step=NS)
    def _(t):
        base = t * SCH
        pltpu.sync_copy(ni_hbm.at[pl.ds(base, SCH)], idxn_v)
        pltpu.sync_copy(ci_hbm.at[pl.ds(base, SCH)], idxc_v)
        pltpu.sync_copy(sn_hbm.at[pl.ds(base, SCH)], bn_v)
        pltpu.sync_copy(sc_hbm.at[pl.ds(base, SCH)], bc_v)
        _compute_local_idx(idxn_v, lin_v, core)
        _compute_local_idx(idxc_v, lic_v, core)
        pltpu.sync_copy(bn_v, acc_sh.at[lin_v], add=True)
        pltpu.sync_copy(bc_v, acc_sh.at[lic_v], add=True)

    plsc.subcore_barrier()

    @pl.loop(sid, 125, step=NS)
    def _(c):
        r0 = c * 200
        pltpu.sync_copy(acc_sh.at[pl.ds(r0, 200)],
                        out_hbm.at[pl.ds(core * A_HALF + r0, 200)])


def _full(shape):
    return pl.BlockSpec(shape, lambda i: (0,) * len(shape))


def _coeff_call(species, p):
    specs = [pl.BlockSpec((BA, 1), lambda i: (i, 0))]
    args = [species]
    for pre, wfw in (("center", NCON), ("neigh", 3 * NWAVE)):
        for nm, shp in (("W0", (1, 8)), ("b0", (1, 8)), ("W1", (8, 8)),
                        ("b1", (1, 8)), ("Wf", (8, wfw)), ("bf", (1, wfw))):
            a = p[pre + "_" + nm].reshape(shp)
            specs.append(_full(shp))
            args.append(a)
    return pl.pallas_call(
        _coeff_body,
        grid=(N_ABLK,),
        in_specs=specs,
        out_specs=[pl.BlockSpec((BA, NCON), lambda i: (i, 0)),
                   pl.BlockSpec((BA, 32), lambda i: (i, 0))],
        out_shape=[jax.ShapeDtypeStruct((N_ATOMS, NCON), jnp.float32),
                   jax.ShapeDtypeStruct((N_ATOMS, 32), jnp.float32)],
    )(*args)


def _edge_fwd_call(cart_c, cart_n, emb):
    return pl.pallas_call(
        _edge_fwd_body,
        grid=(N_EBLK,),
        in_specs=[pl.BlockSpec((BE, 16), lambda i: (i, 0)),
                  pl.BlockSpec((BE, 16), lambda i: (i, 0)),
                  pl.BlockSpec((BE, 32), lambda i: (i, 0))],
        out_specs=pl.BlockSpec((BE, 80), lambda i: (i, 0)),
        out_shape=jax.ShapeDtypeStruct((E_PAD, 80), jnp.float32),
    )(cart_c, cart_n, emb)


def _edge_bwd_call(cart_c, cart_n, emb, gorb):
    return pl.pallas_call(
        _edge_bwd_body,
        grid=(N_EBLK,),
        in_specs=[pl.BlockSpec((BE, 16), lambda i: (i, 0)),
                  pl.BlockSpec((BE, 16), lambda i: (i, 0)),
                  pl.BlockSpec((BE, 32), lambda i: (i, 0)),
                  pl.BlockSpec((BE, 80), lambda i: (i, 0))],
        out_specs=[pl.BlockSpec((BE, 16), lambda i: (i, 0)),
                   pl.BlockSpec((BE, 16), lambda i: (i, 0))],
        out_shape=[jax.ShapeDtypeStruct((E_PAD, 16), jnp.float32),
                   jax.ShapeDtypeStruct((E_PAD, 16), jnp.float32)],
    )(cart_c, cart_n, emb, gorb)


def _atom_call(center_orbital, local_coeff, cc, p):
    args = [center_orbital, local_coeff, cc]
    specs = [pl.BlockSpec((BA, 80), lambda i: (i, 0)),
             pl.BlockSpec((BA, NCON), lambda i: (i, 0)),
             _full((72, NCON))]
    for nm, shp in (("W0", (NCON, 64)), ("b0", (1, 64)), ("W1", (64, 64)),
                    ("b1", (1, 64)), ("Wf", (64, 1)), ("bf", (1, 1))):
        args.append(p["out_" + nm].reshape(shp))
        specs.append(_full(shp))
    return pl.pallas_call(
        _atom_body,
        grid=(N_ABLK,),
        in_specs=specs,
        out_specs=[pl.BlockSpec((BA, 1), lambda i: (i, 0)),
                   pl.BlockSpec((BA, 80), lambda i: (i, 0)),
                   pl.BlockSpec((1, 1), lambda i: (0, 0))],
        out_shape=[jax.ShapeDtypeStruct((N_ATOMS, 1), jnp.float32),
                   jax.ShapeDtypeStruct((N_ATOMS, 80), jnp.float32),
                   jax.ShapeDtypeStruct((1, 1), jnp.float32)],
    )(*args)


def _gather3_call(cart_pad, coeff_pad, ci, ni, si):
    f = pl.kernel(
        _sc_gather3,
        out_type=[jax.ShapeDtypeStruct((E_PAD, 16), jnp.float32),
                  jax.ShapeDtypeStruct((E_PAD, 16), jnp.float32),
                  jax.ShapeDtypeStruct((E_PAD, 32), jnp.float32)],
        mesh=_mesh(),
        compiler_params=_SC_PARAMS,
        scratch_types=[pltpu.VMEM((CHUNK,), jnp.int32),
                       pltpu.VMEM((CHUNK,), jnp.int32),
                       pltpu.VMEM((CHUNK,), jnp.int32),
                       pltpu.VMEM((CHUNK, 16), jnp.float32),
                       pltpu.VMEM((CHUNK, 16), jnp.float32),
                       pltpu.VMEM((CHUNK, 32), jnp.float32),
                       pltpu.SemaphoreType.DMA,
                       pltpu.SemaphoreType.DMA,
                       pltpu.SemaphoreType.DMA],
    )
    return f(cart_pad, coeff_pad, ci, ni, si)


def _gather1_call(table, ci):
    f = pl.kernel(
        _sc_gather1,
        out_type=jax.ShapeDtypeStruct((E_PAD, 80), jnp.float32),
        mesh=_mesh(),
        compiler_params=_SC_PARAMS,
        scratch_types=[pltpu.VMEM((CHUNK,), jnp.int32),
                       pltpu.VMEM((CHUNK, 80), jnp.float32),
                       pltpu.SemaphoreType.DMA],
    )
    return f(table, ci)


def _scatter_orb_call(orb, si):
    f = pl.kernel(
        _sc_scatter_orb,
        out_type=jax.ShapeDtypeStruct((N_ATOMS, 80), jnp.float32),
        mesh=_mesh(),
        compiler_params=_SC_PARAMS,
        scratch_types=[pltpu.VMEM_SHARED((A_ROWS, 80), jnp.float32),
                       pltpu.VMEM((SCH,), jnp.int32),
                       pltpu.VMEM((SCH,), jnp.int32),
                       pltpu.VMEM((SCH, 80), jnp.float32)],
    )
    return f(orb, si)


def _scatter_force_call(sn, sc, ni, ci):
    f = pl.kernel(
        _sc_scatter_force,
        out_type=jax.ShapeDtypeStruct((N_ATOMS, 16), jnp.float32),
        mesh=_mesh(),
        compiler_params=_SC_PARAMS,
        scratch_types=[pltpu.VMEM_SHARED((A_ROWS, 16), jnp.float32),
                       pltpu.VMEM((SCH,), jnp.int32),
                       pltpu.VMEM((SCH,), jnp.int32),
                       pltpu.VMEM((SCH,), jnp.int32),
                       pltpu.VMEM((SCH,), jnp.int32),
                       pltpu.VMEM((SCH, 16), jnp.float32),
                       pltpu.VMEM((SCH, 16), jnp.float32)],
    )
    return f(sn, sc, ni, ci)


def kernel(cart, centerlist, neighlist, local_species, neigh_species, nlocal,
           atom_species, params):
    pad_e = E_PAD - N_EDGES
    cart_pad = jnp.concatenate(
        [cart, jnp.zeros((N_ATOMS, 13), jnp.float32)], axis=1)

    def pad_idx(a, fill):
        return jnp.concatenate(
            [a.astype(jnp.int32), jnp.full((pad_e,), fill, jnp.int32)])

    ci_g = pad_idx(centerlist, 0)
    ni_g = pad_idx(neighlist, 0)
    si_g = pad_idx(local_species, 0)
    ci_s = pad_idx(centerlist, N_ATOMS)
    ni_s = pad_idx(neighlist, N_ATOMS)

    cc = params["contracted_coeff"][0][jnp.array(INDEX_L)].reshape(72, NCON)

    local_coeff, neigh_emb = _coeff_call(atom_species, params)
    cart_c, cart_n, emb = _gather3_call(cart_pad, neigh_emb, ci_g, ni_g, si_g)
    orb = _edge_fwd_call(cart_c, cart_n, emb)
    center_orbital = _scatter_orb_call(orb, ci_s)
    output, g_co, energy = _atom_call(center_orbital, local_coeff, cc, params)
    g_orb = _gather1_call(g_co, ci_g)
    sn, sc_ = _edge_bwd_call(cart_c, cart_n, emb, g_orb)
    neg_grad = _scatter_force_call(sn, sc_, ni_s, ci_s)

    force = neg_grad[:, 0:3].reshape(-1)
    return (energy.reshape(()), force, output)


# speedup vs baseline: 6.7347x; 1.0401x over previous
import functools as _ft

import jax
import jax.numpy as jnp
import numpy as np
from jax import lax
from jax.experimental import pallas as pl
from jax.experimental.pallas import tpu as pltpu
from jax.experimental.pallas import tpu_sc as plsc

N_ATOMS = 50000
N_EDGES = 800000
NWAVE = 8
NANG = 9
NCON = 64
CUTOFF = 4.0
INDEX_L = (0, 1, 1, 1, 2, 2, 2, 2, 2)

NC = 2
NS = 16
LANES = 16
NW = NC * NS

CHUNK = 128
E_PAD = 802816
N_CHUNKS = E_PAD // CHUNK
A_HALF = N_ATOMS // NC
A_ROWS = 25024
TRASH = A_HALF

BA = 2000
BE = 2048
N_ABLK = N_ATOMS // BA
N_EBLK = E_PAD // BE


def _act(x):
    return x * jax.nn.sigmoid(x)


def _layer_norm(x):
    m = jnp.mean(x, axis=-1, keepdims=True)
    v = jnp.var(x, axis=-1, keepdims=True)
    return (x - m) * lax.rsqrt(v + 1e-5)


def _coeff_body(sp_ref, cw0, cb0, cw1, cb1, cwf, cbf, nw0, nb0, nw1, nb1, nwf,
                nbf, lc_ref, emb_ref):
    x = sp_ref[...]

    def mlp(w0, b0, w1, b1, wf, bf):
        h = _act(_layer_norm(x * w0[...] + b0[...]))
        h = h + _act(_layer_norm(
            jnp.dot(h, w1[...], preferred_element_type=jnp.float32) + b1[...]))
        return jnp.dot(h, wf[...], preferred_element_type=jnp.float32) + bf[...]

    lc_ref[...] = mlp(cw0, cb0, cw1, cb1, cwf, cbf)
    nb = mlp(nw0, nb0, nw1, nb1, nwf, nbf)
    emb_ref[...] = jnp.concatenate(
        [nb, jnp.zeros((nb.shape[0], 8), jnp.float32)], axis=1)


def _edge_orbital(dv, emb):
    x = dv[:, 0:1]
    y = dv[:, 1:2]
    z = dv[:, 2:3]
    r2 = x * x + y * y + z * z
    r = jnp.sqrt(r2)
    a = emb[:, 0:NWAVE]
    b = emb[:, NWAVE:2 * NWAVE]
    c0 = emb[:, 2 * NWAVE:3 * NWAVE]
    cut = jnp.square(0.5 * jnp.cos(r * (np.pi / CUTOFF)) + 0.5)
    radial = jnp.exp(-jnp.square(b * (r - c0)))
    rad8 = cut * radial * a
    sph = [
        0.28209479177387814 * jnp.ones_like(x),
        0.4886025119029199 * y,
        0.4886025119029199 * z,
        0.4886025119029199 * x,
        1.0925484305920792 * x * y,
        1.0925484305920792 * y * z,
        0.31539156525252005 * (3.0 * z * z - r2),
        1.0925484305920792 * x * z,
        0.5462742152960396 * (x * x - y * y),
    ]
    return jnp.concatenate([s * rad8 for s in sph], axis=1)


def _edge_fwd_body(dv_ref, emb_ref, orb_ref):
    orb_ref[...] = _edge_orbital(dv_ref[:, 0:3], emb_ref[...])


def _edge_bwd_body(dv_ref, emb_ref, gorb_ref, sn_ref, sc_ref):
    emb = emb_ref[...]
    _, vjp_fn = jax.vjp(lambda d: _edge_orbital(d, emb), dv_ref[:, 0:3])
    (g_dv,) = vjp_fn(gorb_ref[...])
    pad = jnp.zeros((g_dv.shape[0], 13), jnp.float32)
    sn_ref[...] = jnp.concatenate([-g_dv, pad], axis=1)
    sc_ref[...] = jnp.concatenate([g_dv, pad], axis=1)


def _atom_body(co_ref, lc_ref, cc_ref, w0, b0, w1, b1, wf, bf,
               out_ref, gco_ref, en_ref):
    lc = lc_ref[...]
    cc = cc_ref[...]

    def f(co72):
        s = jnp.zeros((co72.shape[0], NCON), jnp.float32)
        for k in range(NANG):
            ck = jnp.dot(co72[:, 8 * k:8 * k + 8], cc[8 * k:8 * k + 8, :],
                         preferred_element_type=jnp.float32)
            s = s + ck * ck
        dens = s * lc
        h = _act(_layer_norm(
            jnp.dot(dens, w0[...], preferred_element_type=jnp.float32) + b0[...]))
        h = h + _act(_layer_norm(
            jnp.dot(h, w1[...], preferred_element_type=jnp.float32) + b1[...]))
        outp = jnp.dot(h, wf[...], preferred_element_type=jnp.float32) + bf[...]
        return jnp.sum(outp), outp

    (e_part, outp), vjp_fn = jax.vjp(f, co_ref[...])
    (g_co,) = vjp_fn((jnp.float32(1.0), jnp.zeros_like(outp)))
    out_ref[...] = outp
    gco_ref[...] = g_co

    @pl.when(pl.program_id(0) == 0)
    def _():
        en_ref[...] = jnp.zeros((1, 1), jnp.float32)

    en_ref[...] = en_ref[...] + jnp.reshape(e_part, (1, 1))


@_ft.cache
def _mesh():
    return plsc.VectorSubcoreMesh(core_axis_name="c", subcore_axis_name="s")


_SC_PARAMS = pltpu.CompilerParams(use_tc_tiling_on_sc=False)


def _sc_gather_dv(cart_hbm, coeff_hbm, ci_hbm, ni_hbm, si_hbm,
                  outdv_hbm, oute_hbm,
                  ic_v, in_v, is_v, bc_v, bn_v, be_v, sem0, sem1, sem2):
    wid = lax.axis_index("s") * NC + lax.axis_index("c")

    @pl.loop(wid, N_CHUNKS, step=NW)
    def _(t):
        base = t * CHUNK
        pltpu.sync_copy(ci_hbm.at[pl.ds(base, CHUNK)], ic_v)
        pltpu.sync_copy(ni_hbm.at[pl.ds(base, CHUNK)], in_v)
        pltpu.sync_copy(si_hbm.at[pl.ds(base, CHUNK)], is_v)
        d0 = pltpu.async_copy(cart_hbm.at[ic_v], bc_v, sem0)
        d1 = pltpu.async_copy(cart_hbm.at[in_v], bn_v, sem1)
        d2 = pltpu.async_copy(coeff_hbm.at[is_v], be_v, sem2)
        d0.wait()
        d1.wait()

        @pl.loop(0, CHUNK)
        def _(r):
            a = bn_v[pl.ds(r, 1), pl.ds(0, LANES)]
            b = bc_v[pl.ds(r, 1), pl.ds(0, LANES)]
            bn_v[pl.ds(r, 1), pl.ds(0, LANES)] = a - b

        d2.wait()
        pltpu.sync_copy(bn_v, outdv_hbm.at[pl.ds(base, CHUNK)])
        pltpu.sync_copy(be_v, oute_hbm.at[pl.ds(base, CHUNK)])


def _sc_gather1(tab_hbm, ci_hbm, out_hbm, i_v, b_v, sem0):
    wid = lax.axis_index("s") * NC + lax.axis_index("c")

    @pl.loop(wid, N_CHUNKS, step=NW)
    def _(t):
        base = t * CHUNK
        pltpu.sync_copy(ci_hbm.at[pl.ds(base, CHUNK)], i_v)
        pltpu.async_copy(tab_hbm.at[i_v], b_v, sem0).wait()
        pltpu.sync_copy(b_v, out_hbm.at[pl.ds(base, CHUNK)])


def _zero_vmem(zv, width):
    zrow = jnp.zeros((1, LANES), jnp.float32)

    @pl.loop(0, zv.shape[0])
    def _(r):
        for c in range(width // LANES):
            zv[pl.ds(r, 1), pl.ds(c * LANES, LANES)] = zrow


def _compute_local_idx(idx_v, li_v, core):
    base = core * A_HALF
    for i in range(CHUNK // LANES):
        v = idx_v[pl.ds(i * LANES, LANES)]
        li = v - base
        ok = (li >= 0) & (li < A_HALF)
        li_v[pl.ds(i * LANES, LANES)] = jnp.where(ok, li, TRASH)


def _make_sc_scatter(width, n_rows):
    n_chunks = n_rows // CHUNK

    def body(st_hbm, si_hbm, out_hbm, acc_sh, idx_v, li_v, buf_v):
        core = lax.axis_index("c")
        sid = lax.axis_index("s")
        _zero_vmem(buf_v, width)

        @pl.loop(sid, A_ROWS // CHUNK, step=NS)
        def _(c):
            pltpu.sync_copy(buf_v, acc_sh.at[pl.ds(c * CHUNK, CHUNK)])

        plsc.subcore_barrier()

        @pl.loop(sid, n_chunks, step=NS)
        def _(t):
            base = t * CHUNK
            pltpu.sync_copy(si_hbm.at[pl.ds(base, CHUNK)], idx_v)
            pltpu.sync_copy(st_hbm.at[pl.ds(base, CHUNK)], buf_v)
            _compute_local_idx(idx_v, li_v, core)
            pltpu.sync_copy(buf_v, acc_sh.at[li_v], add=True)

        plsc.subcore_barrier()

        @pl.loop(sid, 125, step=NS)
        def _(c):
            r0 = c * 200
            pltpu.sync_copy(acc_sh.at[pl.ds(r0, 200)],
                            out_hbm.at[pl.ds(core * A_HALF + r0, 200)])

    def call(stream, sidx):
        f = pl.kernel(
            body,
            out_type=jax.ShapeDtypeStruct((N_ATOMS, width), jnp.float32),
            mesh=_mesh(),
            compiler_params=_SC_PARAMS,
            scratch_types=[pltpu.VMEM_SHARED((A_ROWS, width), jnp.float32),
                           pltpu.VMEM((CHUNK,), jnp.int32),
                           pltpu.VMEM((CHUNK,), jnp.int32),
                           pltpu.VMEM((CHUNK, width), jnp.float32)],
        )
        return f(stream, sidx)

    return call


_scatter_orb_call = _make_sc_scatter(72, E_PAD)
_scatter_force_call = _make_sc_scatter(16, 2 * E_PAD)


def _full(shape):
    return pl.BlockSpec(shape, lambda i: (0,) * len(shape))


def _coeff_call(species, p):
    specs = [pl.BlockSpec((BA, 1), lambda i: (i, 0))]
    args = [species]
    for pre, wfw in (("center", NCON), ("neigh", 3 * NWAVE)):
        for nm, shp in (("W0", (1, 8)), ("b0", (1, 8)), ("W1", (8, 8)),
                        ("b1", (1, 8)), ("Wf", (8, wfw)), ("bf", (1, wfw))):
            a = p[pre + "_" + nm].reshape(shp)
            specs.append(_full(shp))
            args.append(a)
    return pl.pallas_call(
        _coeff_body,
        grid=(N_ABLK,),
        in_specs=specs,
        out_specs=[pl.BlockSpec((BA, NCON), lambda i: (i, 0)),
                   pl.BlockSpec((BA, 32), lambda i: (i, 0))],
        out_shape=[jax.ShapeDtypeStruct((N_ATOMS, NCON), jnp.float32),
                   jax.ShapeDtypeStruct((N_ATOMS, 32), jnp.float32)],
    )(*args)


def _edge_fwd_call(dv, emb):
    return pl.pallas_call(
        _edge_fwd_body,
        grid=(N_EBLK,),
        in_specs=[pl.BlockSpec((BE, 16), lambda i: (i, 0)),
                  pl.BlockSpec((BE, 32), lambda i: (i, 0))],
        out_specs=pl.BlockSpec((BE, 72), lambda i: (i, 0)),
        out_shape=jax.ShapeDtypeStruct((E_PAD, 72), jnp.float32),
    )(dv, emb)


def _edge_bwd_call(dv, emb, gorb):
    return pl.pallas_call(
        _edge_bwd_body,
        grid=(N_EBLK,),
        in_specs=[pl.BlockSpec((BE, 16), lambda i: (i, 0)),
                  pl.BlockSpec((BE, 32), lambda i: (i, 0)),
                  pl.BlockSpec((BE, 72), lambda i: (i, 0))],
        out_specs=[pl.BlockSpec((BE, 16), lambda i: (i, 0)),
                   pl.BlockSpec((BE, 16), lambda i: (i, 0))],
        out_shape=[jax.ShapeDtypeStruct((E_PAD, 16), jnp.float32),
                   jax.ShapeDtypeStruct((E_PAD, 16), jnp.float32)],
    )(dv, emb, gorb)


def _atom_call(center_orbital, local_coeff, cc, p):
    args = [center_orbital, local_coeff, cc]
    specs = [pl.BlockSpec((BA, 72), lambda i: (i, 0)),
             pl.BlockSpec((BA, NCON), lambda i: (i, 0)),
             _full((72, NCON))]
    for nm, shp in (("W0", (NCON, 64)), ("b0", (1, 64)), ("W1", (64, 64)),
                    ("b1", (1, 64)), ("Wf", (64, 1)), ("bf", (1, 1))):
        args.append(p["out_" + nm].reshape(shp))
        specs.append(_full(shp))
    return pl.pallas_call(
        _atom_body,
        grid=(N_ABLK,),
        in_specs=specs,
        out_specs=[pl.BlockSpec((BA, 1), lambda i: (i, 0)),
                   pl.BlockSpec((BA, 72), lambda i: (i, 0)),
                   pl.BlockSpec((1, 1), lambda i: (0, 0))],
        out_shape=[jax.ShapeDtypeStruct((N_ATOMS, 1), jnp.float32),
                   jax.ShapeDtypeStruct((N_ATOMS, 72), jnp.float32),
                   jax.ShapeDtypeStruct((1, 1), jnp.float32)],
    )(*args)


def _gather_dv_call(cart_pad, coeff_pad, ci, ni, si):
    f = pl.kernel(
        _sc_gather_dv,
        out_type=[jax.ShapeDtypeStruct((E_PAD, 16), jnp.float32),
                  jax.ShapeDtypeStruct((E_PAD, 32), jnp.float32)],
        mesh=_mesh(),
        compiler_params=_SC_PARAMS,
        scratch_types=[pltpu.VMEM((CHUNK,), jnp.int32),
                       pltpu.VMEM((CHUNK,), jnp.int32),
                       pltpu.VMEM((CHUNK,), jnp.int32),
                       pltpu.VMEM((CHUNK, 16), jnp.float32),
                       pltpu.VMEM((CHUNK, 16), jnp.float32),
                       pltpu.VMEM((CHUNK, 32), jnp.float32),
                       pltpu.SemaphoreType.DMA,
                       pltpu.SemaphoreType.DMA,
                       pltpu.SemaphoreType.DMA],
    )
    return f(cart_pad, coeff_pad, ci, ni, si)


def _gather1_call(table, ci):
    f = pl.kernel(
        _sc_gather1,
        out_type=jax.ShapeDtypeStruct((E_PAD, 72), jnp.float32),
        mesh=_mesh(),
        compiler_params=_SC_PARAMS,
        scratch_types=[pltpu.VMEM((CHUNK,), jnp.int32),
                       pltpu.VMEM((CHUNK, 72), jnp.float32),
                       pltpu.SemaphoreType.DMA],
    )
    return f(table, ci)


def kernel(cart, centerlist, neighlist, local_species, neigh_species, nlocal,
           atom_species, params):
    pad_e = E_PAD - N_EDGES
    cart_pad = jnp.concatenate(
        [cart, jnp.zeros((N_ATOMS, 13), jnp.float32)], axis=1)

    def pad_idx(a, fill):
        return jnp.concatenate(
            [a.astype(jnp.int32), jnp.full((pad_e,), fill, jnp.int32)])

    ci_g = pad_idx(centerlist, 0)
    ni_g = pad_idx(neighlist, 0)
    si_g = pad_idx(local_species, 0)
    ci_s = pad_idx(centerlist, N_ATOMS)
    ni_s = pad_idx(neighlist, N_ATOMS)

    cc = params["contracted_coeff"][0][jnp.array(INDEX_L)].reshape(72, NCON)

    local_coeff, neigh_emb = _coeff_call(atom_species, params)
    dv, emb = _gather_dv_call(cart_pad, neigh_emb, ci_g, ni_g, si_g)
    orb = _edge_fwd_call(dv, emb)
    center_orbital = _scatter_orb_call(orb, ci_s)
    output, g_co, energy = _atom_call(center_orbital, local_coeff, cc, params)
    g_orb = _gather1_call(g_co, ci_g)
    sn, sc_ = _edge_bwd_call(dv, emb, g_orb)
    force_stream = jnp.concatenate([sn, sc_], axis=0)
    force_idx = jnp.concatenate([ni_s, ci_s], axis=0)
    neg_grad = _scatter_force_call(force_stream, force_idx)

    force = neg_grad[:, 0:3].reshape(-1)
    return (energy.reshape(()), force, output)


# speedup vs baseline: 18.7334x; 2.7816x over previous
import functools as _ft

import jax
import jax.numpy as jnp
import numpy as np
from jax import lax
from jax.experimental import pallas as pl
from jax.experimental.pallas import tpu as pltpu
from jax.experimental.pallas import tpu_sc as plsc

N_ATOMS = 50000
N_EDGES = 800000
NWAVE = 8
NANG = 9
NCON = 64
CUTOFF = 4.0
INDEX_L = (0, 1, 1, 1, 2, 2, 2, 2, 2)

NC = 2
NS = 16
LANES = 16
NW = NC * NS

CHUNK = 128
E_PAD = 802816
N_CHUNKS = E_PAD // CHUNK
A_HALF = N_ATOMS // NC
A_ROWS = 25024
TRASH = A_HALF

BA = 2000
BE = 8192
N_ABLK = N_ATOMS // BA
N_EBLK = E_PAD // BE


def _act(x):
    return x * jax.nn.sigmoid(x)


def _layer_norm(x):
    m = jnp.mean(x, axis=-1, keepdims=True)
    v = jnp.var(x, axis=-1, keepdims=True)
    return (x - m) * lax.rsqrt(v + 1e-5)


def _coeff_body(sp_ref, cw0, cb0, cw1, cb1, cwf, cbf, nw0, nb0, nw1, nb1, nwf,
                nbf, lc_ref, emb_ref):
    x = sp_ref[...]

    def mlp(w0, b0, w1, b1, wf, bf):
        h = _act(_layer_norm(x * w0[...] + b0[...]))
        h = h + _act(_layer_norm(
            jnp.dot(h, w1[...], preferred_element_type=jnp.float32) + b1[...]))
        return jnp.dot(h, wf[...], preferred_element_type=jnp.float32) + bf[...]

    lc_ref[...] = mlp(cw0, cb0, cw1, cb1, cwf, cbf)
    nb = mlp(nw0, nb0, nw1, nb1, nwf, nbf)
    emb_ref[...] = jnp.concatenate(
        [nb, jnp.zeros((nb.shape[0], 8), jnp.float32)], axis=1)


def _edge_orbital(dv, emb):
    x = dv[:, 0:1]
    y = dv[:, 1:2]
    z = dv[:, 2:3]
    r2 = x * x + y * y + z * z
    r = jnp.sqrt(r2)
    a = emb[:, 0:NWAVE]
    b = emb[:, NWAVE:2 * NWAVE]
    c0 = emb[:, 2 * NWAVE:3 * NWAVE]
    cut = jnp.square(0.5 * jnp.cos(r * (np.pi / CUTOFF)) + 0.5)
    radial = jnp.exp(-jnp.square(b * (r - c0)))
    rad8 = cut * radial * a
    sph = [
        0.28209479177387814 * jnp.ones_like(x),
        0.4886025119029199 * y,
        0.4886025119029199 * z,
        0.4886025119029199 * x,
        1.0925484305920792 * x * y,
        1.0925484305920792 * y * z,
        0.31539156525252005 * (3.0 * z * z - r2),
        1.0925484305920792 * x * z,
        0.5462742152960396 * (x * x - y * y),
    ]
    return jnp.concatenate([s * rad8 for s in sph], axis=1)


def _edge_orbital_t(dvt, embt):
    x = dvt[0:1, :]
    y = dvt[1:2, :]
    z = dvt[2:3, :]
    r2 = x * x + y * y + z * z
    r = jnp.sqrt(r2)
    a = embt[0:NWAVE, :]
    b = embt[NWAVE:2 * NWAVE, :]
    c0 = embt[2 * NWAVE:3 * NWAVE, :]
    cut = jnp.square(0.5 * jnp.cos(r * (np.pi / CUTOFF)) + 0.5)
    radial = jnp.exp(-jnp.square(b * (r - c0)))
    rad8 = cut * radial * a
    sph = [
        0.28209479177387814 * jnp.ones_like(x),
        0.4886025119029199 * y,
        0.4886025119029199 * z,
        0.4886025119029199 * x,
        1.0925484305920792 * x * y,
        1.0925484305920792 * y * z,
        0.31539156525252005 * (3.0 * z * z - r2),
        1.0925484305920792 * x * z,
        0.5462742152960396 * (x * x - y * y),
    ]
    return jnp.concatenate([s * rad8 for s in sph], axis=0)


def _edge_fwd_body(dvt_ref, embt_ref, orbt_ref):
    orbt_ref[...] = _edge_orbital_t(dvt_ref[0:3, :], embt_ref[...])


def _edge_bwd_body(dvt_ref, embt_ref, gorbt_ref, snt_ref, sct_ref):
    embt = embt_ref[...]
    _, vjp_fn = jax.vjp(lambda d: _edge_orbital_t(d, embt), dvt_ref[0:3, :])
    (g_dv,) = vjp_fn(gorbt_ref[...])
    pad = jnp.zeros((13, g_dv.shape[1]), jnp.float32)
    snt_ref[...] = jnp.concatenate([-g_dv, pad], axis=0)
    sct_ref[...] = jnp.concatenate([g_dv, pad], axis=0)


def _atom_body(co_ref, lc_ref, cc_ref, w0, b0, w1, b1, wf, bf,
               out_ref, gco_ref, en_ref):
    lc = lc_ref[...]
    cc = cc_ref[...]

    def f(co72):
        s = jnp.zeros((co72.shape[0], NCON), jnp.float32)
        for k in range(NANG):
            ck = jnp.dot(co72[:, 8 * k:8 * k + 8], cc[8 * k:8 * k + 8, :],
                         preferred_element_type=jnp.float32)
            s = s + ck * ck
        dens = s * lc
        h = _act(_layer_norm(
            jnp.dot(dens, w0[...], preferred_element_type=jnp.float32) + b0[...]))
        h = h + _act(_layer_norm(
            jnp.dot(h, w1[...], preferred_element_type=jnp.float32) + b1[...]))
        outp = jnp.dot(h, wf[...], preferred_element_type=jnp.float32) + bf[...]
        return jnp.sum(outp), outp

    (e_part, outp), vjp_fn = jax.vjp(f, co_ref[...])
    (g_co,) = vjp_fn((jnp.float32(1.0), jnp.zeros_like(outp)))
    out_ref[...] = outp
    gco_ref[...] = g_co

    @pl.when(pl.program_id(0) == 0)
    def _():
        en_ref[...] = jnp.zeros((1, 1), jnp.float32)

    en_ref[...] = en_ref[...] + jnp.reshape(e_part, (1, 1))


@_ft.cache
def _mesh():
    return plsc.VectorSubcoreMesh(core_axis_name="c", subcore_axis_name="s")


_SC_PARAMS = pltpu.CompilerParams(use_tc_tiling_on_sc=False)


def _sc_gather_dv(cart_hbm, coeff_hbm, ci_hbm, ni_hbm, si_hbm,
                  outdv_hbm, oute_hbm,
                  ic_v, in_v, is_v, bc_v, bn_v, be_v, sem0, sem1, sem2):
    wid = lax.axis_index("s") * NC + lax.axis_index("c")

    @pl.loop(wid, N_CHUNKS, step=NW)
    def _(t):
        base = t * CHUNK
        pltpu.sync_copy(ci_hbm.at[pl.ds(base, CHUNK)], ic_v)
        pltpu.sync_copy(ni_hbm.at[pl.ds(base, CHUNK)], in_v)
        pltpu.sync_copy(si_hbm.at[pl.ds(base, CHUNK)], is_v)
        d0 = pltpu.async_copy(cart_hbm.at[ic_v], bc_v, sem0)
        d1 = pltpu.async_copy(cart_hbm.at[in_v], bn_v, sem1)
        d2 = pltpu.async_copy(coeff_hbm.at[is_v], be_v, sem2)
        d0.wait()
        d1.wait()

        @pl.loop(0, CHUNK)
        def _(r):
            a = bn_v[pl.ds(r, 1), pl.ds(0, LANES)]
            b = bc_v[pl.ds(r, 1), pl.ds(0, LANES)]
            bn_v[pl.ds(r, 1), pl.ds(0, LANES)] = a - b

        d2.wait()
        pltpu.sync_copy(bn_v, outdv_hbm.at[pl.ds(base, CHUNK)])
        pltpu.sync_copy(be_v, oute_hbm.at[pl.ds(base, CHUNK)])


def _sc_gather1(tab_hbm, ci_hbm, out_hbm, i_v, b_v, sem0):
    wid = lax.axis_index("s") * NC + lax.axis_index("c")

    @pl.loop(wid, N_CHUNKS, step=NW)
    def _(t):
        base = t * CHUNK
        pltpu.sync_copy(ci_hbm.at[pl.ds(base, CHUNK)], i_v)
        pltpu.async_copy(tab_hbm.at[i_v], b_v, sem0).wait()
        pltpu.sync_copy(b_v, out_hbm.at[pl.ds(base, CHUNK)])


def _zero_vmem(zv, width):
    zrow = jnp.zeros((1, LANES), jnp.float32)

    @pl.loop(0, zv.shape[0])
    def _(r):
        for c in range(width // LANES):
            zv[pl.ds(r, 1), pl.ds(c * LANES, LANES)] = zrow


def _compute_local_idx(idx_v, li_v, core):
    base = core * A_HALF
    for i in range(CHUNK // LANES):
        v = idx_v[pl.ds(i * LANES, LANES)]
        li = v - base
        ok = (li >= 0) & (li < A_HALF)
        li_v[pl.ds(i * LANES, LANES)] = jnp.where(ok, li, TRASH)


def _make_sc_scatter(width, n_rows):
    n_chunks = n_rows // CHUNK

    def body(st_hbm, si_hbm, out_hbm, acc_sh, idx_v, li_v, buf_v):
        core = lax.axis_index("c")
        sid = lax.axis_index("s")
        _zero_vmem(buf_v, width)

        @pl.loop(sid, A_ROWS // CHUNK, step=NS)
        def _(c):
            pltpu.sync_copy(buf_v, acc_sh.at[pl.ds(c * CHUNK, CHUNK)])

        plsc.subcore_barrier()

        @pl.loop(sid, n_chunks, step=NS)
        def _(t):
            base = t * CHUNK
            pltpu.sync_copy(si_hbm.at[pl.ds(base, CHUNK)], idx_v)
            pltpu.sync_copy(st_hbm.at[pl.ds(base, CHUNK)], buf_v)
            _compute_local_idx(idx_v, li_v, core)
            pltpu.sync_copy(buf_v, acc_sh.at[li_v], add=True)

        plsc.subcore_barrier()

        @pl.loop(sid, 125, step=NS)
        def _(c):
            r0 = c * 200
            pltpu.sync_copy(acc_sh.at[pl.ds(r0, 200)],
                            out_hbm.at[pl.ds(core * A_HALF + r0, 200)])

    def call(stream, sidx):
        f = pl.kernel(
            body,
            out_type=jax.ShapeDtypeStruct((N_ATOMS, width), jnp.float32),
            mesh=_mesh(),
            compiler_params=_SC_PARAMS,
            scratch_types=[pltpu.VMEM_SHARED((A_ROWS, width), jnp.float32),
                           pltpu.VMEM((CHUNK,), jnp.int32),
                           pltpu.VMEM((CHUNK,), jnp.int32),
                           pltpu.VMEM((CHUNK, width), jnp.float32)],
        )
        return f(stream, sidx)

    return call


_scatter_orb_call = _make_sc_scatter(72, E_PAD)
_scatter_force_call = _make_sc_scatter(16, 2 * E_PAD)


def _full(shape):
    return pl.BlockSpec(shape, lambda i: (0,) * len(shape))


def _coeff_call(species, p):
    specs = [pl.BlockSpec((BA, 1), lambda i: (i, 0))]
    args = [species]
    for pre, wfw in (("center", NCON), ("neigh", 3 * NWAVE)):
        for nm, shp in (("W0", (1, 8)), ("b0", (1, 8)), ("W1", (8, 8)),
                        ("b1", (1, 8)), ("Wf", (8, wfw)), ("bf", (1, wfw))):
            a = p[pre + "_" + nm].reshape(shp)
            specs.append(_full(shp))
            args.append(a)
    return pl.pallas_call(
        _coeff_body,
        grid=(N_ABLK,),
        in_specs=specs,
        out_specs=[pl.BlockSpec((BA, NCON), lambda i: (i, 0)),
                   pl.BlockSpec((BA, 32), lambda i: (i, 0))],
        out_shape=[jax.ShapeDtypeStruct((N_ATOMS, NCON), jnp.float32),
                   jax.ShapeDtypeStruct((N_ATOMS, 32), jnp.float32)],
    )(*args)


def _edge_fwd_call(dvt, embt):
    return pl.pallas_call(
        _edge_fwd_body,
        grid=(N_EBLK,),
        in_specs=[pl.BlockSpec((16, BE), lambda i: (0, i)),
                  pl.BlockSpec((32, BE), lambda i: (0, i))],
        out_specs=pl.BlockSpec((72, BE), lambda i: (0, i)),
        out_shape=jax.ShapeDtypeStruct((72, E_PAD), jnp.float32),
    )(dvt, embt)


def _edge_bwd_call(dvt, embt, gorbt):
    return pl.pallas_call(
        _edge_bwd_body,
        grid=(N_EBLK,),
        in_specs=[pl.BlockSpec((16, BE), lambda i: (0, i)),
                  pl.BlockSpec((32, BE), lambda i: (0, i)),
                  pl.BlockSpec((72, BE), lambda i: (0, i))],
        out_specs=[pl.BlockSpec((16, BE), lambda i: (0, i)),
                   pl.BlockSpec((16, BE), lambda i: (0, i))],
        out_shape=[jax.ShapeDtypeStruct((16, E_PAD), jnp.float32),
                   jax.ShapeDtypeStruct((16, E_PAD), jnp.float32)],
    )(dvt, embt, gorbt)


def _atom_call(center_orbital, local_coeff, cc, p):
    args = [center_orbital, local_coeff, cc]
    specs = [pl.BlockSpec((BA, 72), lambda i: (i, 0)),
             pl.BlockSpec((BA, NCON), lambda i: (i, 0)),
             _full((72, NCON))]
    for nm, shp in (("W0", (NCON, 64)), ("b0", (1, 64)), ("W1", (64, 64)),
                    ("b1", (1, 64)), ("Wf", (64, 1)), ("bf", (1, 1))):
        args.append(p["out_" + nm].reshape(shp))
        specs.append(_full(shp))
    return pl.pallas_call(
        _atom_body,
        grid=(N_ABLK,),
        in_specs=specs,
        out_specs=[pl.BlockSpec((BA, 1), lambda i: (i, 0)),
                   pl.BlockSpec((BA, 72), lambda i: (i, 0)),
                   pl.BlockSpec((1, 1), lambda i: (0, 0))],
        out_shape=[jax.ShapeDtypeStruct((N_ATOMS, 1), jnp.float32),
                   jax.ShapeDtypeStruct((N_ATOMS, 72), jnp.float32),
                   jax.ShapeDtypeStruct((1, 1), jnp.float32)],
    )(*args)


def _gather_dv_call(cart_pad, coeff_pad, ci, ni, si):
    f = pl.kernel(
        _sc_gather_dv,
        out_type=[jax.ShapeDtypeStruct((E_PAD, 16), jnp.float32),
                  jax.ShapeDtypeStruct((E_PAD, 32), jnp.float32)],
        mesh=_mesh(),
        compiler_params=_SC_PARAMS,
        scratch_types=[pltpu.VMEM((CHUNK,), jnp.int32),
                       pltpu.VMEM((CHUNK,), jnp.int32),
                       pltpu.VMEM((CHUNK,), jnp.int32),
                       pltpu.VMEM((CHUNK, 16), jnp.float32),
                       pltpu.VMEM((CHUNK, 16), jnp.float32),
                       pltpu.VMEM((CHUNK, 32), jnp.float32),
                       pltpu.SemaphoreType.DMA,
                       pltpu.SemaphoreType.DMA,
                       pltpu.SemaphoreType.DMA],
    )
    return f(cart_pad, coeff_pad, ci, ni, si)


def _gather1_call(table, ci):
    f = pl.kernel(
        _sc_gather1,
        out_type=jax.ShapeDtypeStruct((E_PAD, 72), jnp.float32),
        mesh=_mesh(),
        compiler_params=_SC_PARAMS,
        scratch_types=[pltpu.VMEM((CHUNK,), jnp.int32),
                       pltpu.VMEM((CHUNK, 72), jnp.float32),
                       pltpu.SemaphoreType.DMA],
    )
    return f(table, ci)


def kernel(cart, centerlist, neighlist, local_species, neigh_species, nlocal,
           atom_species, params):
    pad_e = E_PAD - N_EDGES
    cart_pad = jnp.concatenate(
        [cart, jnp.zeros((N_ATOMS, 13), jnp.float32)], axis=1)

    def pad_idx(a, fill):
        return jnp.concatenate(
            [a.astype(jnp.int32), jnp.full((pad_e,), fill, jnp.int32)])

    ci_g = pad_idx(centerlist, 0)
    ni_g = pad_idx(neighlist, 0)
    si_g = pad_idx(local_species, 0)
    ci_s = pad_idx(centerlist, N_ATOMS)
    ni_s = pad_idx(neighlist, N_ATOMS)

    cc = params["contracted_coeff"][0][jnp.array(INDEX_L)].reshape(72, NCON)

    local_coeff, neigh_emb = _coeff_call(atom_species, params)
    dv, emb = _gather_dv_call(cart_pad, neigh_emb, ci_g, ni_g, si_g)
    dvt = jnp.transpose(dv, (1, 0))
    embt = jnp.transpose(emb, (1, 0))
    orbt = _edge_fwd_call(dvt, embt)
    center_orbital = _scatter_orb_call(jnp.transpose(orbt, (1, 0)), ci_s)
    output, g_co, energy = _atom_call(center_orbital, local_coeff, cc, params)
    g_orb = _gather1_call(g_co, ci_g)
    snt, sct = _edge_bwd_call(dvt, embt, jnp.transpose(g_orb, (1, 0)))
    sn = jnp.transpose(snt, (1, 0))
    sc_ = jnp.transpose(sct, (1, 0))
    force_stream = jnp.concatenate([sn, sc_], axis=0)
    force_idx = jnp.concatenate([ni_s, ci_s], axis=0)
    neg_grad = _scatter_force_call(force_stream, force_idx)

    force = neg_grad[:, 0:3].reshape(-1)
    return (energy.reshape(()), force, output)
